# Initial kernel scaffold; baseline (speedup 1.0000x reference)
import functools

import jax
import jax.numpy as jnp
from jax import lax
from jax.experimental import pallas as pl
from jax.experimental.pallas import tpu as pltpu
from jax.experimental.pallas import tpu_sc as plsc

N = 10000
E = 320000
D_IN = 128
H1 = 32
H2 = 64

NC = 2
NS = 16
NW = NC * NS

NP = 10240
EP = 323584
ROWS = EP // 128
RW = ROWS // NW
NPW = NP // NS

_MESH = plsc.VectorSubcoreMesh(core_axis_name="c", subcore_axis_name="s")


@functools.partial(
    pl.kernel,
    out_type=jax.ShapeDtypeStruct((NC, NP), jnp.float32),
    mesh=_MESH,
    scratch_types=[
        pltpu.VMEM((RW, 128), jnp.int32),
        pltpu.VMEM((128,), jnp.float32),
        pltpu.VMEM((NPW,), jnp.float32),
        pltpu.VMEM_SHARED((NP,), jnp.float32),
    ],
)
def _sc_degree(dst_hbm, out_hbm, didx, ones, zeros, acc):
    c = lax.axis_index("c")
    s = lax.axis_index("s")
    w = s * NC + c

    for i in range(8):
        ones[pl.ds(i * 16, 16)] = jnp.ones((16,), jnp.float32)
    for i in range(NPW // 16):
        zeros[pl.ds(i * 16, 16)] = jnp.zeros((16,), jnp.float32)
    pltpu.sync_copy(zeros, acc.at[pl.ds(s * NPW, NPW)])
    plsc.subcore_barrier()

    pltpu.sync_copy(dst_hbm.at[pl.ds(w * RW, RW), :], didx)

    @pl.loop(0, RW)
    def _(j):
        pltpu.sync_copy(ones, acc.at[didx.at[j]], add=True)

    plsc.subcore_barrier()
    pltpu.sync_copy(acc.at[pl.ds(s * NPW, NPW)], out_hbm.at[c, pl.ds(s * NPW, NPW)])


def _make_sc_propagate(width):
    @functools.partial(
        pl.kernel,
        out_type=jax.ShapeDtypeStruct((NC, NP, width), jnp.float32),
        mesh=_MESH,
        scratch_types=[
            pltpu.VMEM((RW, 128), jnp.int32),
            pltpu.VMEM((RW, 128), jnp.int32),
            pltpu.VMEM((128, width), jnp.float32),
            pltpu.VMEM_SHARED((NP, width), jnp.float32),
            pltpu.SemaphoreType.DMA,
        ],
    )
    def prop(y_hbm, src_hbm, dst_hbm, out_hbm, sidx, didx, rows, acc, sem):
        c = lax.axis_index("c")
        s = lax.axis_index("s")
        w = s * NC + c

        pltpu.sync_copy(y_hbm.at[pl.ds(s * NPW, NPW), :], acc.at[pl.ds(s * NPW, NPW), :])
        plsc.subcore_barrier()

        pltpu.sync_copy(src_hbm.at[pl.ds(w * RW, RW), :], sidx)
        pltpu.sync_copy(dst_hbm.at[pl.ds(w * RW, RW), :], didx)

        @pl.loop(0, RW)
        def _(j):
            pltpu.async_copy(y_hbm.at[sidx.at[j]], rows, sem).wait()
            pltpu.sync_copy(rows, acc.at[didx.at[j]], add=True)

        plsc.subcore_barrier()
        pltpu.sync_copy(
            acc.at[pl.ds(s * NPW, NPW), :], out_hbm.at[c, pl.ds(s * NPW, NPW), :]
        )

    return prop


_sc_prop32 = _make_sc_propagate(H1)
_sc_prop64 = _make_sc_propagate(H2)


_BR = 1024


def _tc1_body(x_ref, w1_ref, p0_ref, p1_ref, y_ref, dis_ref):
    deg = p0_ref[...] + p1_ref[...] + 1.0
    dis = lax.rsqrt(deg)
    xw = jnp.dot(x_ref[...], w1_ref[...], preferred_element_type=jnp.float32)
    y_ref[...] = xw * dis
    dis_ref[...] = dis


def _tc1(x_p, W1, p0, p1):
    return pl.pallas_call(
        _tc1_body,
        grid=(NP // _BR,),
        in_specs=[
            pl.BlockSpec((_BR, D_IN), lambda i: (i, 0)),
            pl.BlockSpec((D_IN, H1), lambda i: (0, 0)),
            pl.BlockSpec((_BR, 1), lambda i: (i, 0)),
            pl.BlockSpec((_BR, 1), lambda i: (i, 0)),
        ],
        out_specs=[
            pl.BlockSpec((_BR, H1), lambda i: (i, 0)),
            pl.BlockSpec((_BR, 1), lambda i: (i, 0)),
        ],
        out_shape=[
            jax.ShapeDtypeStruct((NP, H1), jnp.float32),
            jax.ShapeDtypeStruct((NP, 1), jnp.float32),
        ],
    )(x_p, W1, p0, p1)


def _tc2_body(a0_ref, a1_ref, y1_ref, dis_ref, w2_ref, b1_ref, y2_ref):
    dis = dis_ref[...]
    h1 = jnp.maximum(dis * (a0_ref[...] + a1_ref[...] - y1_ref[...]) + b1_ref[...], 0.0)
    y2_ref[...] = jnp.dot(h1, w2_ref[...], preferred_element_type=jnp.float32) * dis


def _tc2(a0, a1, y1, dis, W2, b1):
    return pl.pallas_call(
        _tc2_body,
        grid=(NP // _BR,),
        in_specs=[
            pl.BlockSpec((_BR, H1), lambda i: (i, 0)),
            pl.BlockSpec((_BR, H1), lambda i: (i, 0)),
            pl.BlockSpec((_BR, H1), lambda i: (i, 0)),
            pl.BlockSpec((_BR, 1), lambda i: (i, 0)),
            pl.BlockSpec((H1, H2), lambda i: (0, 0)),
            pl.BlockSpec((1, H1), lambda i: (0, 0)),
        ],
        out_specs=pl.BlockSpec((_BR, H2), lambda i: (i, 0)),
        out_shape=jax.ShapeDtypeStruct((NP, H2), jnp.float32),
    )(a0, a1, y1, dis, W2, b1)


_BR3 = 2000


def _tc3_body(a0_ref, a1_ref, y2_ref, dis_ref, b2_ref, wo_ref, bo_ref, o_ref):
    dis = dis_ref[...]
    h2 = jnp.maximum(dis * (a0_ref[...] + a1_ref[...] - y2_ref[...]) + b2_ref[...], 0.0)
    part = jnp.sum(h2 * wo_ref[...])

    @pl.when(pl.program_id(0) == 0)
    def _():
        o_ref[...] = bo_ref[...]

    o_ref[0, 0] += part


def _tc3(a0, a1, y2, dis, b2, Wo, bo):
    return pl.pallas_call(
        _tc3_body,
        grid=(N // _BR3,),
        in_specs=[
            pl.BlockSpec((_BR3, H2), lambda i: (i, 0)),
            pl.BlockSpec((_BR3, H2), lambda i: (i, 0)),
            pl.BlockSpec((_BR3, H2), lambda i: (i, 0)),
            pl.BlockSpec((_BR3, 1), lambda i: (i, 0)),
            pl.BlockSpec((1, H2), lambda i: (0, 0)),
            pl.BlockSpec((_BR3, H2), lambda i: (i, 0)),
            pl.BlockSpec((1, 1), lambda i: (0, 0)),
        ],
        out_specs=pl.BlockSpec((1, 1), lambda i: (0, 0)),
        out_shape=jax.ShapeDtypeStruct((1, 1), jnp.float32),
    )(a0, a1, y2, dis, b2, Wo, bo)


def kernel(x, edge_index, W1, b1, W2, b2, W_out, b_out):
    src = edge_index[0]
    dst = edge_index[1]
    pad = jnp.full((EP - E,), N, dtype=jnp.int32)
    src2 = jnp.concatenate([src, pad]).reshape(ROWS, 128)
    dst2 = jnp.concatenate([dst, pad]).reshape(ROWS, 128)
    x_p = jnp.concatenate([x, jnp.zeros((NP - N, D_IN), jnp.float32)])

    degp = _sc_degree(dst2)
    p0 = degp[0].reshape(NP, 1)
    p1 = degp[1].reshape(NP, 1)

    y1, dis = _tc1(x_p, W1, p0, p1)
    acc1 = _sc_prop32(y1, src2, dst2)
    y2 = _tc2(acc1[0], acc1[1], y1, dis, W2, b1.reshape(1, H1))
    acc2 = _sc_prop64(y2, src2, dst2)
    out = _tc3(
        acc2[0], acc2[1], y2, dis,
        b2.reshape(1, H2), W_out.reshape(N, H2), b_out.reshape(1, 1),
    )
    return out


# speedup vs baseline: 18.3630x; 18.3630x over previous
import functools

import jax
import jax.numpy as jnp
from jax import lax
from jax.experimental import pallas as pl
from jax.experimental.pallas import tpu as pltpu
from jax.experimental.pallas import tpu_sc as plsc

N = 10000
E = 320000
D_IN = 128
H1 = 32
H2 = 64

NC = 2
NS = 16
NW = NC * NS

NP = 10240
EP = 327680
ROWS = EP // 128
RW = ROWS // NW
NPW = NP // NS

_MESH = plsc.VectorSubcoreMesh(core_axis_name="c", subcore_axis_name="s")
_SC_PARAMS = pltpu.CompilerParams(use_tc_tiling_on_sc=False)


@functools.partial(
    pl.kernel,
    out_type=jax.ShapeDtypeStruct((NC, NP), jnp.float32),
    mesh=_MESH,
    compiler_params=_SC_PARAMS,
    scratch_types=[
        pltpu.VMEM((RW, 128), jnp.int32),
        pltpu.VMEM((128,), jnp.float32),
        pltpu.VMEM((NPW,), jnp.float32),
        pltpu.VMEM_SHARED((NP,), jnp.float32),
    ],
)
def _sc_degree(dst_hbm, out_hbm, didx, ones, zeros, acc):
    c = lax.axis_index("c")
    s = lax.axis_index("s")
    w = s * NC + c

    for i in range(8):
        ones[pl.ds(i * 16, 16)] = jnp.ones((16,), jnp.float32)
    for i in range(NPW // 16):
        zeros[pl.ds(i * 16, 16)] = jnp.zeros((16,), jnp.float32)
    pltpu.sync_copy(zeros, acc.at[pl.ds(s * NPW, NPW)])
    plsc.subcore_barrier()

    pltpu.sync_copy(dst_hbm.at[pl.ds(w * RW, RW), :], didx)

    @pl.loop(0, RW)
    def _(j):
        pltpu.sync_copy(ones, acc.at[didx.at[j]], add=True)

    plsc.subcore_barrier()
    pltpu.sync_copy(acc.at[pl.ds(s * NPW, NPW)], out_hbm.at[c, pl.ds(s * NPW, NPW)])


def _make_sc_propagate(width):
    @functools.partial(
        pl.kernel,
        out_type=jax.ShapeDtypeStruct((NC, NP, width), jnp.float32),
        mesh=_MESH,
        compiler_params=_SC_PARAMS,
        scratch_types=[
            pltpu.VMEM((RW, 128), jnp.int32),
            pltpu.VMEM((RW, 128), jnp.int32),
            pltpu.VMEM((128, width), jnp.float32),
            pltpu.VMEM_SHARED((NP, width), jnp.float32),
            pltpu.SemaphoreType.DMA,
        ],
    )
    def prop(y_hbm, src_hbm, dst_hbm, out_hbm, sidx, didx, rows, acc, sem):
        c = lax.axis_index("c")
        s = lax.axis_index("s")
        w = s * NC + c

        pltpu.sync_copy(y_hbm.at[pl.ds(s * NPW, NPW), :], acc.at[pl.ds(s * NPW, NPW), :])
        plsc.subcore_barrier()

        pltpu.sync_copy(src_hbm.at[pl.ds(w * RW, RW), :], sidx)
        pltpu.sync_copy(dst_hbm.at[pl.ds(w * RW, RW), :], didx)

        @pl.loop(0, RW)
        def _(j):
            pltpu.async_copy(y_hbm.at[sidx.at[j]], rows, sem).wait()
            pltpu.sync_copy(rows, acc.at[didx.at[j]], add=True)

        plsc.subcore_barrier()
        pltpu.sync_copy(
            acc.at[pl.ds(s * NPW, NPW), :], out_hbm.at[c, pl.ds(s * NPW, NPW), :]
        )

    return prop


_sc_prop32 = _make_sc_propagate(H1)
_sc_prop64 = _make_sc_propagate(H2)


_BR = 1024


def _tc1_body(x_ref, w1_ref, p0_ref, p1_ref, y_ref, dis_ref):
    deg = p0_ref[...] + p1_ref[...] + 1.0
    dis = lax.rsqrt(deg)
    xw = jnp.dot(x_ref[...], w1_ref[...], preferred_element_type=jnp.float32)
    y_ref[...] = xw * dis
    dis_ref[...] = dis


def _tc1(x_p, W1, p0, p1):
    return pl.pallas_call(
        _tc1_body,
        grid=(NP // _BR,),
        in_specs=[
            pl.BlockSpec((_BR, D_IN), lambda i: (i, 0)),
            pl.BlockSpec((D_IN, H1), lambda i: (0, 0)),
            pl.BlockSpec((_BR, 1), lambda i: (i, 0)),
            pl.BlockSpec((_BR, 1), lambda i: (i, 0)),
        ],
        out_specs=[
            pl.BlockSpec((_BR, H1), lambda i: (i, 0)),
            pl.BlockSpec((_BR, 1), lambda i: (i, 0)),
        ],
        out_shape=[
            jax.ShapeDtypeStruct((NP, H1), jnp.float32),
            jax.ShapeDtypeStruct((NP, 1), jnp.float32),
        ],
    )(x_p, W1, p0, p1)


def _tc2_body(a0_ref, a1_ref, y1_ref, dis_ref, w2_ref, b1_ref, y2_ref):
    dis = dis_ref[...]
    h1 = jnp.maximum(dis * (a0_ref[...] + a1_ref[...] - y1_ref[...]) + b1_ref[...], 0.0)
    y2_ref[...] = jnp.dot(h1, w2_ref[...], preferred_element_type=jnp.float32) * dis


def _tc2(a0, a1, y1, dis, W2, b1):
    return pl.pallas_call(
        _tc2_body,
        grid=(NP // _BR,),
        in_specs=[
            pl.BlockSpec((_BR, H1), lambda i: (i, 0)),
            pl.BlockSpec((_BR, H1), lambda i: (i, 0)),
            pl.BlockSpec((_BR, H1), lambda i: (i, 0)),
            pl.BlockSpec((_BR, 1), lambda i: (i, 0)),
            pl.BlockSpec((H1, H2), lambda i: (0, 0)),
            pl.BlockSpec((1, H1), lambda i: (0, 0)),
        ],
        out_specs=pl.BlockSpec((_BR, H2), lambda i: (i, 0)),
        out_shape=jax.ShapeDtypeStruct((NP, H2), jnp.float32),
    )(a0, a1, y1, dis, W2, b1)


_BR3 = 2000


def _tc3_body(a0_ref, a1_ref, y2_ref, dis_ref, b2_ref, wo_ref, bo_ref, o_ref):
    dis = dis_ref[...]
    h2 = jnp.maximum(dis * (a0_ref[...] + a1_ref[...] - y2_ref[...]) + b2_ref[...], 0.0)
    part = jnp.sum(h2 * wo_ref[...], keepdims=True)

    @pl.when(pl.program_id(0) == 0)
    def _():
        o_ref[...] = bo_ref[...]

    o_ref[...] += part


def _tc3(a0, a1, y2, dis, b2, Wo, bo):
    return pl.pallas_call(
        _tc3_body,
        grid=(N // _BR3,),
        in_specs=[
            pl.BlockSpec((_BR3, H2), lambda i: (i, 0)),
            pl.BlockSpec((_BR3, H2), lambda i: (i, 0)),
            pl.BlockSpec((_BR3, H2), lambda i: (i, 0)),
            pl.BlockSpec((_BR3, 1), lambda i: (i, 0)),
            pl.BlockSpec((1, H2), lambda i: (0, 0)),
            pl.BlockSpec((_BR3, H2), lambda i: (i, 0)),
            pl.BlockSpec((1, 1), lambda i: (0, 0)),
        ],
        out_specs=pl.BlockSpec((1, 1), lambda i: (0, 0)),
        out_shape=jax.ShapeDtypeStruct((1, 1), jnp.float32),
    )(a0, a1, y2, dis, b2, Wo, bo)


def kernel(x, edge_index, W1, b1, W2, b2, W_out, b_out):
    src = edge_index[0]
    dst = edge_index[1]
    pad = jnp.full((EP - E,), N, dtype=jnp.int32)
    src2 = jnp.concatenate([src, pad]).reshape(ROWS, 128)
    dst2 = jnp.concatenate([dst, pad]).reshape(ROWS, 128)
    x_p = jnp.concatenate([x, jnp.zeros((NP - N, D_IN), jnp.float32)])

    degp = _sc_degree(dst2)
    p0 = degp[0].reshape(NP, 1)
    p1 = degp[1].reshape(NP, 1)

    y1, dis = _tc1(x_p, W1, p0, p1)
    acc1 = _sc_prop32(y1, src2, dst2)
    y2 = _tc2(acc1[0], acc1[1], y1, dis, W2, b1.reshape(1, H1))
    acc2 = _sc_prop64(y2, src2, dst2)
    out = _tc3(
        acc2[0], acc2[1], y2, dis,
        b2.reshape(1, H2), W_out.reshape(N, H2), b_out.reshape(1, 1),
    )
    return out


# speedup vs baseline: 32.5274x; 1.7713x over previous
import functools

import jax
import jax.numpy as jnp
from jax import lax
from jax.experimental import pallas as pl
from jax.experimental.pallas import tpu as pltpu
from jax.experimental.pallas import tpu_sc as plsc

N = 10000
E = 320000
D_IN = 128
H1 = 32
H2 = 64

NC = 2
NS = 16
NW = NC * NS

NP = 10240
EP = 327680
ROWS = EP // 128
RW = ROWS // NW
NPW = NP // NS

_MESH = plsc.VectorSubcoreMesh(core_axis_name="c", subcore_axis_name="s")
_SC_PARAMS = pltpu.CompilerParams(use_tc_tiling_on_sc=False)


@functools.partial(
    pl.kernel,
    out_type=jax.ShapeDtypeStruct((NC, NP), jnp.float32),
    mesh=_MESH,
    compiler_params=_SC_PARAMS,
    scratch_types=[
        pltpu.VMEM((RW, 128), jnp.int32),
        pltpu.VMEM((128,), jnp.float32),
        pltpu.VMEM((NPW,), jnp.float32),
        pltpu.VMEM_SHARED((NP,), jnp.float32),
    ],
)
def _sc_degree(dst_hbm, out_hbm, didx, ones, zeros, acc):
    c = lax.axis_index("c")
    s = lax.axis_index("s")
    w = s * NC + c

    for i in range(8):
        ones[pl.ds(i * 16, 16)] = jnp.ones((16,), jnp.float32)
    for i in range(NPW // 16):
        zeros[pl.ds(i * 16, 16)] = jnp.zeros((16,), jnp.float32)
    pltpu.sync_copy(zeros, acc.at[pl.ds(s * NPW, NPW)])
    plsc.subcore_barrier()

    pltpu.sync_copy(dst_hbm.at[pl.ds(w * RW, RW), :], didx)

    @pl.loop(0, RW)
    def _(j):
        pltpu.sync_copy(ones, acc.at[didx.at[j]], add=True)

    plsc.subcore_barrier()
    pltpu.sync_copy(acc.at[pl.ds(s * NPW, NPW)], out_hbm.at[c, pl.ds(s * NPW, NPW)])


def _make_sc_propagate(width):
    @functools.partial(
        pl.kernel,
        out_type=jax.ShapeDtypeStruct((NC, NP, width), jnp.float32),
        mesh=_MESH,
        compiler_params=_SC_PARAMS,
        scratch_types=[
            pltpu.VMEM((RW, 128), jnp.int32),
            pltpu.VMEM((RW, 128), jnp.int32),
            pltpu.VMEM((128, width), jnp.float32),
            pltpu.VMEM_SHARED((NP, width), jnp.float32),
            pltpu.VMEM_SHARED((NP, width), jnp.float32),
            pltpu.SemaphoreType.DMA,
        ],
    )
    def prop(y_hbm, src_hbm, dst_hbm, out_hbm, sidx, didx, rows, y_sp, acc, sem):
        c = lax.axis_index("c")
        s = lax.axis_index("s")
        w = s * NC + c

        pltpu.sync_copy(y_hbm.at[pl.ds(s * NPW, NPW), :], y_sp.at[pl.ds(s * NPW, NPW), :])
        pltpu.sync_copy(y_hbm.at[pl.ds(s * NPW, NPW), :], acc.at[pl.ds(s * NPW, NPW), :])
        plsc.subcore_barrier()

        pltpu.sync_copy(src_hbm.at[pl.ds(w * RW, RW), :], sidx)
        pltpu.sync_copy(dst_hbm.at[pl.ds(w * RW, RW), :], didx)

        @pl.loop(0, RW)
        def _(j):
            pltpu.async_copy(y_sp.at[sidx.at[j]], rows, sem).wait()
            pltpu.sync_copy(rows, acc.at[didx.at[j]], add=True)

        plsc.subcore_barrier()
        pltpu.sync_copy(
            acc.at[pl.ds(s * NPW, NPW), :], out_hbm.at[c, pl.ds(s * NPW, NPW), :]
        )

    return prop


_sc_prop32 = _make_sc_propagate(H1)
_sc_prop64 = _make_sc_propagate(H2)


_BR = 1024


def _tc1_body(x_ref, w1_ref, p0_ref, p1_ref, y_ref, dis_ref):
    deg = p0_ref[...] + p1_ref[...] + 1.0
    dis = lax.rsqrt(deg)
    xw = jnp.dot(x_ref[...], w1_ref[...], preferred_element_type=jnp.float32)
    y_ref[...] = xw * dis
    dis_ref[...] = dis


def _tc1(x_p, W1, p0, p1):
    return pl.pallas_call(
        _tc1_body,
        grid=(NP // _BR,),
        in_specs=[
            pl.BlockSpec((_BR, D_IN), lambda i: (i, 0)),
            pl.BlockSpec((D_IN, H1), lambda i: (0, 0)),
            pl.BlockSpec((_BR, 1), lambda i: (i, 0)),
            pl.BlockSpec((_BR, 1), lambda i: (i, 0)),
        ],
        out_specs=[
            pl.BlockSpec((_BR, H1), lambda i: (i, 0)),
            pl.BlockSpec((_BR, 1), lambda i: (i, 0)),
        ],
        out_shape=[
            jax.ShapeDtypeStruct((NP, H1), jnp.float32),
            jax.ShapeDtypeStruct((NP, 1), jnp.float32),
        ],
    )(x_p, W1, p0, p1)


def _tc2_body(a0_ref, a1_ref, y1_ref, dis_ref, w2_ref, b1_ref, y2_ref):
    dis = dis_ref[...]
    h1 = jnp.maximum(dis * (a0_ref[...] + a1_ref[...] - y1_ref[...]) + b1_ref[...], 0.0)
    y2_ref[...] = jnp.dot(h1, w2_ref[...], preferred_element_type=jnp.float32) * dis


def _tc2(a0, a1, y1, dis, W2, b1):
    return pl.pallas_call(
        _tc2_body,
        grid=(NP // _BR,),
        in_specs=[
            pl.BlockSpec((_BR, H1), lambda i: (i, 0)),
            pl.BlockSpec((_BR, H1), lambda i: (i, 0)),
            pl.BlockSpec((_BR, H1), lambda i: (i, 0)),
            pl.BlockSpec((_BR, 1), lambda i: (i, 0)),
            pl.BlockSpec((H1, H2), lambda i: (0, 0)),
            pl.BlockSpec((1, H1), lambda i: (0, 0)),
        ],
        out_specs=pl.BlockSpec((_BR, H2), lambda i: (i, 0)),
        out_shape=jax.ShapeDtypeStruct((NP, H2), jnp.float32),
    )(a0, a1, y1, dis, W2, b1)


_BR3 = 2000


def _tc3_body(a0_ref, a1_ref, y2_ref, dis_ref, b2_ref, wo_ref, bo_ref, o_ref):
    dis = dis_ref[...]
    h2 = jnp.maximum(dis * (a0_ref[...] + a1_ref[...] - y2_ref[...]) + b2_ref[...], 0.0)
    part = jnp.sum(h2 * wo_ref[...], keepdims=True)

    @pl.when(pl.program_id(0) == 0)
    def _():
        o_ref[...] = bo_ref[...]

    o_ref[...] += part


def _tc3(a0, a1, y2, dis, b2, Wo, bo):
    return pl.pallas_call(
        _tc3_body,
        grid=(N // _BR3,),
        in_specs=[
            pl.BlockSpec((_BR3, H2), lambda i: (i, 0)),
            pl.BlockSpec((_BR3, H2), lambda i: (i, 0)),
            pl.BlockSpec((_BR3, H2), lambda i: (i, 0)),
            pl.BlockSpec((_BR3, 1), lambda i: (i, 0)),
            pl.BlockSpec((1, H2), lambda i: (0, 0)),
            pl.BlockSpec((_BR3, H2), lambda i: (i, 0)),
            pl.BlockSpec((1, 1), lambda i: (0, 0)),
        ],
        out_specs=pl.BlockSpec((1, 1), lambda i: (0, 0)),
        out_shape=jax.ShapeDtypeStruct((1, 1), jnp.float32),
    )(a0, a1, y2, dis, b2, Wo, bo)


def kernel(x, edge_index, W1, b1, W2, b2, W_out, b_out):
    src = edge_index[0]
    dst = edge_index[1]
    pad = jnp.full((EP - E,), N, dtype=jnp.int32)
    src2 = jnp.concatenate([src, pad]).reshape(ROWS, 128)
    dst2 = jnp.concatenate([dst, pad]).reshape(ROWS, 128)
    x_p = jnp.concatenate([x, jnp.zeros((NP - N, D_IN), jnp.float32)])

    degp = _sc_degree(dst2)
    p0 = degp[0].reshape(NP, 1)
    p1 = degp[1].reshape(NP, 1)

    y1, dis = _tc1(x_p, W1, p0, p1)
    acc1 = _sc_prop32(y1, src2, dst2)
    y2 = _tc2(acc1[0], acc1[1], y1, dis, W2, b1.reshape(1, H1))
    acc2 = _sc_prop64(y2, src2, dst2)
    out = _tc3(
        acc2[0], acc2[1], y2, dis,
        b2.reshape(1, H2), W_out.reshape(N, H2), b_out.reshape(1, 1),
    )
    return out
